# baseline (speedup 1.0000x reference)
import functools

import jax
import jax.numpy as jnp
from jax import lax
from jax.experimental import pallas as pl
from jax.experimental.pallas import tpu as pltpu
from jax.experimental.pallas import tpu_sc as plsc

N = 10000
E = 160000
D = 256

NC = 2
NS = 16
NW = NC * NS


_BN = 2000
_BE = 2000


def _full(shape):
    return pl.BlockSpec(shape, lambda i: (0,) * len(shape))


def _rows(shape):
    return pl.BlockSpec(shape, lambda i: (i,) + (0,) * (len(shape) - 1))


def _proj_body(x_ref, w_ref, out_ref):
    d = jnp.dot(x_ref[...], w_ref[...], preferred_element_type=jnp.float32, precision=lax.Precision.HIGHEST)
    out_ref[0] = d[:, :D]
    out_ref[1] = d[:, D:2 * D]
    out_ref[2] = d[:, 2 * D:]


def _proj(x, w):
    return pl.pallas_call(
        _proj_body,
        grid=(N // _BN,),
        in_specs=[_rows((_BN, D)), _full((D, 3 * D))],
        out_specs=pl.BlockSpec((3, _BN, D), lambda i: (0, i, 0)),
        out_shape=jax.ShapeDtypeStruct((3, N, D), jnp.float32),
    )(x, w)


def _edge2_body(gs_ref, gd_ref, e_ref, w1e_ref, b1_ref, w2_ref, b2_ref,
                out_ref):
    h = jnp.maximum(gs_ref[...] + gd_ref[...] + b1_ref[...]
                    + jnp.dot(e_ref[...], w1e_ref[...],
                              preferred_element_type=jnp.float32, precision=lax.Precision.HIGHEST),
                    0.0)
    out_ref[...] = (jnp.dot(h, w2_ref[...],
                            preferred_element_type=jnp.float32, precision=lax.Precision.HIGHEST)
                    + b2_ref[...] + e_ref[...])


def _edge2(gs, gd, e, w1e, b1, w2, b2):
    return pl.pallas_call(
        _edge2_body,
        grid=(E // _BE,),
        in_specs=[_rows((_BE, D)), _rows((_BE, D)), _rows((_BE, D)),
                  _full((D, D)), _full((1, D)), _full((D, D)),
                  _full((1, D))],
        out_specs=_rows((_BE, D)),
        out_shape=jax.ShapeDtypeStruct((E, D), jnp.float32),
    )(gs, gd, e, w1e, b1, w2, b2)


def _node2_body(p_ref, agg_ref, x_ref, w1a_ref, b1_ref, w2_ref, b2_ref,
                out_ref):
    t = (p_ref[0] + b1_ref[...]
         + jnp.dot(agg_ref[...], w1a_ref[...],
                   preferred_element_type=jnp.float32, precision=lax.Precision.HIGHEST))
    h = jnp.maximum(t, 0.0)
    out_ref[...] = (jnp.dot(h, w2_ref[...],
                            preferred_element_type=jnp.float32, precision=lax.Precision.HIGHEST)
                    + b2_ref[...] + x_ref[...])


def _node2(p, agg, x, w1a, b1, w2, b2):
    return pl.pallas_call(
        _node2_body,
        grid=(N // _BN,),
        in_specs=[pl.BlockSpec((1, _BN, D), lambda i: (2, i, 0)),
                  _rows((_BN, D)), _rows((_BN, D)),
                  _full((D, D)), _full((1, D)), _full((D, D)), _full((1, D))],
        out_specs=_rows((_BN, D)),
        out_shape=jax.ShapeDtypeStruct((N, D), jnp.float32),
    )(p, agg, x, w1a, b1, w2, b2)


def _out_mlp_body(x_ref, w1_ref, b1_ref, w2_ref, b2_ref, w3_ref, b3_ref,
                  out_ref):
    h = jnp.maximum(jnp.dot(x_ref[...], w1_ref[...],
                            preferred_element_type=jnp.float32, precision=lax.Precision.HIGHEST) + b1_ref[...],
                    0.0)
    h = jnp.maximum(jnp.dot(h, w2_ref[...],
                            preferred_element_type=jnp.float32, precision=lax.Precision.HIGHEST) + b2_ref[...],
                    0.0)
    out_ref[...] = jnp.dot(h, w3_ref[...],
                           preferred_element_type=jnp.float32, precision=lax.Precision.HIGHEST) + b3_ref[...]


def _out_mlp(x, w1, b1, w2, b2, w3p, b3p):
    return pl.pallas_call(
        _out_mlp_body,
        grid=(N // _BN,),
        in_specs=[_rows((_BN, D)),
                  _full((D, D)), _full((1, D)),
                  _full((D, D)), _full((1, D)),
                  _full((D, 128)), _full((1, 128))],
        out_specs=_rows((_BN, 128)),
        out_shape=jax.ShapeDtypeStruct((N, 128), jnp.float32),
    )(x, w1, b1, w2, b2, w3p, b3p)


_GC = 200
_EPW = E // NW
_NCK = _EPW // _GC


@functools.lru_cache(maxsize=None)
def _make_sc_gather2():
    mesh = plsc.VectorSubcoreMesh(core_axis_name="c", subcore_axis_name="s")

    @functools.partial(
        pl.kernel,
        out_type=[jax.ShapeDtypeStruct((E, D), jnp.float32),
                  jax.ShapeDtypeStruct((E, D), jnp.float32)],
        mesh=mesh,
        scratch_types=[
            pltpu.VMEM((_GC,), jnp.int32),
            pltpu.VMEM((_GC,), jnp.int32),
            pltpu.VMEM((_GC,), jnp.int32),
            pltpu.VMEM((_GC,), jnp.int32),
            pltpu.VMEM((_GC, D), jnp.float32),
            pltpu.VMEM((_GC, D), jnp.float32),
            pltpu.SemaphoreType.DMA,
            pltpu.SemaphoreType.DMA,
            pltpu.SemaphoreType.DMA,
            pltpu.SemaphoreType.DMA,
        ],
    )
    def sc_gather2(p_hbm, src_hbm, dst_hbm, gs_hbm, gd_hbm,
                   si0, di0, si1, di1, rsS, rsD,
                   semS, semD, soS, soD):
        wid = lax.axis_index("s") * NC + lax.axis_index("c")
        base = wid * _EPW

        def idx_load(c, si, di):
            off = base + c * _GC
            pltpu.sync_copy(src_hbm.at[pl.ds(off, _GC)], si)
            pltpu.sync_copy(dst_hbm.at[pl.ds(off, _GC)], di)

        def step(c, iP, iQ, first, last):
            si, di = iP
            if not first:
                pltpu.make_async_copy(rsS, gs_hbm.at[pl.ds(0, _GC)],
                                      soS).wait()
            pltpu.async_copy(p_hbm.at[si], rsS, semS)
            if not first:
                pltpu.make_async_copy(rsD, gd_hbm.at[pl.ds(0, _GC)],
                                      soD).wait()
            pltpu.async_copy(p_hbm.at[di], rsD, semD)
            if not last:
                idx_load(c + 1, iQ[0], iQ[1])
            off = base + c * _GC
            pltpu.make_async_copy(p_hbm.at[si], rsS, semS).wait()
            pltpu.async_copy(rsS, gs_hbm.at[pl.ds(off, _GC)], soS)
            pltpu.make_async_copy(p_hbm.at[di], rsD, semD).wait()
            pltpu.async_copy(rsD, gd_hbm.at[pl.ds(off, _GC)], soD)

        i0 = (si0, di0)
        i1 = (si1, di1)
        idx_load(0, si0, di0)
        step(0, i0, i1, first=True, last=False)

        def pair(i, carry):
            c = 1 + 2 * i
            step(c, i1, i0, first=False, last=False)
            step(c + 1, i0, i1, first=False, last=False)
            return carry

        lax.fori_loop(0, (_NCK - 3) // 2, pair, 0)

        step(_NCK - 2, i1, i0, first=False, last=False)
        step(_NCK - 1, i0, i1, first=False, last=True)
        pltpu.make_async_copy(rsS, gs_hbm.at[pl.ds(0, _GC)], soS).wait()
        pltpu.make_async_copy(rsD, gd_hbm.at[pl.ds(0, _GC)], soD).wait()

    return sc_gather2


def _sc_gather2(p, src, dstN):
    return _make_sc_gather2()(p, src, dstN)


_NP = 10240
_SC_NPC = _NP // NS
_SCC = 80
_EPS = E // NS
_SNCK = _EPS // _SCC
_HD = D // NC


@functools.lru_cache(maxsize=None)
def _make_sc_scatter_add():
    mesh = plsc.VectorSubcoreMesh(core_axis_name="c", subcore_axis_name="s")

    @functools.partial(
        pl.kernel,
        out_type=jax.ShapeDtypeStruct((_NP, D), jnp.float32),
        mesh=mesh,
        scratch_types=[
            pltpu.VMEM((_SCC,), jnp.int32),
            pltpu.VMEM((_SCC,), jnp.int32),
            pltpu.VMEM((_SCC, _HD), jnp.float32),
            pltpu.VMEM((_SCC, _HD), jnp.float32),
            pltpu.VMEM_SHARED((_NP, _HD), jnp.float32),
            pltpu.SemaphoreType.DMA,
            pltpu.SemaphoreType.DMA,
            pltpu.SemaphoreType.DMA,
            pltpu.SemaphoreType.DMA,
        ],
    )
    def sc_scatter_add(enew_hbm, dst_hbm, zero_hbm, agg_hbm,
                       di0, di1, rw0, rw1, acc_sh, sr0, sr1, sa0, sa1):
        c = lax.axis_index("c")
        s = lax.axis_index("s")
        pltpu.sync_copy(zero_hbm, acc_sh.at[pl.ds(s * _SC_NPC, _SC_NPC)])
        plsc.subcore_barrier()

        col = c * _HD
        base = s * _EPS
        b0 = (di0, rw0, sr0, sa0)
        b1 = (di1, rw1, sr1, sa1)

        def load(ck, buf):
            off = base + ck * _SCC
            pltpu.sync_copy(dst_hbm.at[pl.ds(off, _SCC)], buf[0])
            pltpu.async_copy(enew_hbm.at[pl.ds(off, _SCC), pl.ds(col, _HD)],
                             buf[1], buf[2])

        def add(buf):
            di, rw, sr, sa = buf
            pltpu.make_async_copy(
                enew_hbm.at[pl.ds(0, _SCC), pl.ds(col, _HD)], rw, sr).wait()
            pltpu.async_copy(rw, acc_sh.at[di], sa, add=True)

        def wait_add(buf):
            pltpu.make_async_copy(buf[1], acc_sh.at[buf[0]], buf[3]).wait()

        def step(ck, bP, bQ, wait_p):
            if wait_p:
                wait_add(bP)
            load(ck, bP)
            add(bQ)

        load(0, b0)
        step(1, b1, b0, wait_p=False)

        def pair(i, carry):
            ck = 2 + 2 * i
            step(ck, b0, b1, wait_p=True)
            step(ck + 1, b1, b0, wait_p=True)
            return carry

        lax.fori_loop(0, (_SNCK - 2) // 2, pair, 0)

        if _SNCK % 2 == 1:
            step(_SNCK - 1, b0, b1, wait_p=True)
            add(b0)
            wait_add(b1)
            wait_add(b0)
        else:
            add(b1)
            wait_add(b0)
            wait_add(b1)

        plsc.subcore_barrier()
        pltpu.sync_copy(
            acc_sh.at[pl.ds(s * _SC_NPC, _SC_NPC)],
            agg_hbm.at[pl.ds(s * _SC_NPC, _SC_NPC), pl.ds(col, _HD)])

    return sc_scatter_add


def _sc_scatter_add(e_new, dst, zero):
    return _make_sc_scatter_add()(e_new, dst, zero)[:N]


def kernel(x, edge_index, edge_attr, params):
    src = edge_index[0].astype(jnp.int32)
    dst = edge_index[1].astype(jnp.int32)
    dstN = dst + N
    zero = jnp.zeros((_SC_NPC, _HD), jnp.float32)

    for c in range(len(params['convs'])):
        ep = params['convs'][c]['edge_mlp']
        npar = params['convs'][c]['node_mlp']
        (w1, b1), (w2, b2) = ep
        (wn1, bn1), (wn2, bn2) = npar
        w_all = jnp.concatenate([w1[:D], w1[D:2 * D], wn1[:D]], axis=1)
        p = _proj(x, w_all)
        gs, gd = _sc_gather2(p.reshape(3 * N, D), src, dstN)
        e_new = _edge2(gs, gd, edge_attr, w1[2 * D:], b1.reshape(1, D), w2,
                       b2.reshape(1, D))
        agg = _sc_scatter_add(e_new, dst, zero)
        x = _node2(p, agg, x, wn1[D:], bn1.reshape(1, D), wn2,
                   bn2.reshape(1, D))
        edge_attr = e_new

    (wo1, bo1), (wo2, bo2), (wo3, bo3) = params['out']
    out_dim = wo3.shape[1]
    w3p = jnp.pad(wo3, ((0, 0), (0, 128 - out_dim)))
    b3p = jnp.pad(bo3, ((0, 128 - out_dim),))
    o = _out_mlp(x, wo1, bo1.reshape(1, D), wo2, bo2.reshape(1, D),
                 w3p, b3p.reshape(1, 128))
    return o[:, :out_dim]


# speedup vs baseline: 1.5599x; 1.5599x over previous
import functools

import jax
import jax.numpy as jnp
from jax import lax
from jax.experimental import pallas as pl
from jax.experimental.pallas import tpu as pltpu
from jax.experimental.pallas import tpu_sc as plsc

N = 10000
E = 160000
D = 256

NC = 2
NS = 16
NW = NC * NS


_BN = 2000
_BE = 2000


def _full(shape):
    return pl.BlockSpec(shape, lambda i: (0,) * len(shape))


def _rows(shape):
    return pl.BlockSpec(shape, lambda i: (i,) + (0,) * (len(shape) - 1))


def _proj_body(x_ref, w_ref, out_ref):
    d = jnp.dot(x_ref[...], w_ref[...], preferred_element_type=jnp.float32)
    out_ref[0] = d[:, :D]
    out_ref[1] = d[:, D:2 * D]
    out_ref[2] = d[:, 2 * D:]


def _proj(x, w):
    return pl.pallas_call(
        _proj_body,
        grid=(N // _BN,),
        in_specs=[_rows((_BN, D)), _full((D, 3 * D))],
        out_specs=pl.BlockSpec((3, _BN, D), lambda i: (0, i, 0)),
        out_shape=jax.ShapeDtypeStruct((3, N, D), jnp.float32),
    )(x, w)


def _edge2_body(gs_ref, gd_ref, e_ref, w1e_ref, b1_ref, w2_ref, b2_ref,
                out_ref):
    h = jnp.maximum((gs_ref[...] + gd_ref[...]
                     + jnp.dot(e_ref[...], w1e_ref[...],
                               preferred_element_type=jnp.float32))
                    + b1_ref[...],
                    0.0)
    out_ref[...] = (jnp.dot(h, w2_ref[...],
                            preferred_element_type=jnp.float32)
                    + b2_ref[...] + e_ref[...])


def _edge2(gs, gd, e, w1e, b1, w2, b2):
    return pl.pallas_call(
        _edge2_body,
        grid=(E // _BE,),
        in_specs=[_rows((_BE, D)), _rows((_BE, D)), _rows((_BE, D)),
                  _full((D, D)), _full((1, D)), _full((D, D)),
                  _full((1, D))],
        out_specs=_rows((_BE, D)),
        out_shape=jax.ShapeDtypeStruct((E, D), jnp.float32),
    )(gs, gd, e, w1e, b1, w2, b2)


def _node2_body(p_ref, agg_ref, x_ref, w1a_ref, b1_ref, w2_ref, b2_ref,
                out_ref):
    t = (p_ref[0]
         + jnp.dot(agg_ref[...], w1a_ref[...],
                   preferred_element_type=jnp.float32)) + b1_ref[...]
    h = jnp.maximum(t, 0.0)
    out_ref[...] = (jnp.dot(h, w2_ref[...],
                            preferred_element_type=jnp.float32)
                    + b2_ref[...] + x_ref[...])


def _node2(p, agg, x, w1a, b1, w2, b2):
    return pl.pallas_call(
        _node2_body,
        grid=(N // _BN,),
        in_specs=[pl.BlockSpec((1, _BN, D), lambda i: (2, i, 0)),
                  _rows((_BN, D)), _rows((_BN, D)),
                  _full((D, D)), _full((1, D)), _full((D, D)), _full((1, D))],
        out_specs=_rows((_BN, D)),
        out_shape=jax.ShapeDtypeStruct((N, D), jnp.float32),
    )(p, agg, x, w1a, b1, w2, b2)


def _out_mlp_body(x_ref, w1_ref, b1_ref, w2_ref, b2_ref, w3_ref, b3_ref,
                  out_ref):
    h = jnp.maximum(jnp.dot(x_ref[...], w1_ref[...],
                            preferred_element_type=jnp.float32) + b1_ref[...],
                    0.0)
    h = jnp.maximum(jnp.dot(h, w2_ref[...],
                            preferred_element_type=jnp.float32) + b2_ref[...],
                    0.0)
    out_ref[...] = jnp.dot(h, w3_ref[...],
                           preferred_element_type=jnp.float32) + b3_ref[...]


def _out_mlp(x, w1, b1, w2, b2, w3p, b3p):
    return pl.pallas_call(
        _out_mlp_body,
        grid=(N // _BN,),
        in_specs=[_rows((_BN, D)),
                  _full((D, D)), _full((1, D)),
                  _full((D, D)), _full((1, D)),
                  _full((D, 128)), _full((1, 128))],
        out_specs=_rows((_BN, 128)),
        out_shape=jax.ShapeDtypeStruct((N, 128), jnp.float32),
    )(x, w1, b1, w2, b2, w3p, b3p)


_GC = 200
_EPW = E // NW
_NCK = _EPW // _GC


@functools.lru_cache(maxsize=None)
def _make_sc_gather2():
    mesh = plsc.VectorSubcoreMesh(core_axis_name="c", subcore_axis_name="s")

    @functools.partial(
        pl.kernel,
        out_type=[jax.ShapeDtypeStruct((E, D), jnp.float32),
                  jax.ShapeDtypeStruct((E, D), jnp.float32)],
        mesh=mesh,
        scratch_types=[
            pltpu.VMEM((_GC,), jnp.int32),
            pltpu.VMEM((_GC,), jnp.int32),
            pltpu.VMEM((_GC,), jnp.int32),
            pltpu.VMEM((_GC,), jnp.int32),
            pltpu.VMEM((_GC, D), jnp.float32),
            pltpu.VMEM((_GC, D), jnp.float32),
            pltpu.SemaphoreType.DMA,
            pltpu.SemaphoreType.DMA,
            pltpu.SemaphoreType.DMA,
            pltpu.SemaphoreType.DMA,
        ],
    )
    def sc_gather2(p_hbm, src_hbm, dst_hbm, gs_hbm, gd_hbm,
                   si0, di0, si1, di1, rsS, rsD,
                   semS, semD, soS, soD):
        wid = lax.axis_index("s") * NC + lax.axis_index("c")
        base = wid * _EPW

        def idx_load(c, si, di):
            off = base + c * _GC
            pltpu.sync_copy(src_hbm.at[pl.ds(off, _GC)], si)
            pltpu.sync_copy(dst_hbm.at[pl.ds(off, _GC)], di)

        def step(c, iP, iQ, first, last):
            si, di = iP
            if not first:
                pltpu.make_async_copy(rsS, gs_hbm.at[pl.ds(0, _GC)],
                                      soS).wait()
            pltpu.async_copy(p_hbm.at[si], rsS, semS)
            if not first:
                pltpu.make_async_copy(rsD, gd_hbm.at[pl.ds(0, _GC)],
                                      soD).wait()
            pltpu.async_copy(p_hbm.at[di], rsD, semD)
            if not last:
                idx_load(c + 1, iQ[0], iQ[1])
            off = base + c * _GC
            pltpu.make_async_copy(p_hbm.at[si], rsS, semS).wait()
            pltpu.async_copy(rsS, gs_hbm.at[pl.ds(off, _GC)], soS)
            pltpu.make_async_copy(p_hbm.at[di], rsD, semD).wait()
            pltpu.async_copy(rsD, gd_hbm.at[pl.ds(off, _GC)], soD)

        i0 = (si0, di0)
        i1 = (si1, di1)
        idx_load(0, si0, di0)
        step(0, i0, i1, first=True, last=False)

        def pair(i, carry):
            c = 1 + 2 * i
            step(c, i1, i0, first=False, last=False)
            step(c + 1, i0, i1, first=False, last=False)
            return carry

        lax.fori_loop(0, (_NCK - 3) // 2, pair, 0)

        step(_NCK - 2, i1, i0, first=False, last=False)
        step(_NCK - 1, i0, i1, first=False, last=True)
        pltpu.make_async_copy(rsS, gs_hbm.at[pl.ds(0, _GC)], soS).wait()
        pltpu.make_async_copy(rsD, gd_hbm.at[pl.ds(0, _GC)], soD).wait()

    return sc_gather2


def _sc_gather2(p, src, dstN):
    return _make_sc_gather2()(p, src, dstN)


_NP = 10240
_SC_NPC = _NP // NS
_SCC = 80
_EPS = E // NS
_SNCK = _EPS // _SCC
_HD = D // NC


@functools.lru_cache(maxsize=None)
def _make_sc_scatter_add():
    mesh = plsc.VectorSubcoreMesh(core_axis_name="c", subcore_axis_name="s")

    @functools.partial(
        pl.kernel,
        out_type=jax.ShapeDtypeStruct((_NP, D), jnp.float32),
        mesh=mesh,
        scratch_types=[
            pltpu.VMEM((_SCC,), jnp.int32),
            pltpu.VMEM((_SCC,), jnp.int32),
            pltpu.VMEM((_SCC, _HD), jnp.float32),
            pltpu.VMEM((_SCC, _HD), jnp.float32),
            pltpu.VMEM_SHARED((_NP, _HD), jnp.float32),
            pltpu.SemaphoreType.DMA,
            pltpu.SemaphoreType.DMA,
            pltpu.SemaphoreType.DMA,
            pltpu.SemaphoreType.DMA,
        ],
    )
    def sc_scatter_add(enew_hbm, dst_hbm, zero_hbm, agg_hbm,
                       di0, di1, rw0, rw1, acc_sh, sr0, sr1, sa0, sa1):
        c = lax.axis_index("c")
        s = lax.axis_index("s")
        pltpu.sync_copy(zero_hbm, acc_sh.at[pl.ds(s * _SC_NPC, _SC_NPC)])
        plsc.subcore_barrier()

        col = c * _HD
        base = s * _EPS
        b0 = (di0, rw0, sr0, sa0)
        b1 = (di1, rw1, sr1, sa1)

        def load(ck, buf):
            off = base + ck * _SCC
            pltpu.sync_copy(dst_hbm.at[pl.ds(off, _SCC)], buf[0])
            pltpu.async_copy(enew_hbm.at[pl.ds(off, _SCC), pl.ds(col, _HD)],
                             buf[1], buf[2])

        def add(buf):
            di, rw, sr, sa = buf
            pltpu.make_async_copy(
                enew_hbm.at[pl.ds(0, _SCC), pl.ds(col, _HD)], rw, sr).wait()
            pltpu.async_copy(rw, acc_sh.at[di], sa, add=True)

        def wait_add(buf):
            pltpu.make_async_copy(buf[1], acc_sh.at[buf[0]], buf[3]).wait()

        def step(ck, bP, bQ, wait_p):
            if wait_p:
                wait_add(bP)
            load(ck, bP)
            add(bQ)

        load(0, b0)
        step(1, b1, b0, wait_p=False)

        def pair(i, carry):
            ck = 2 + 2 * i
            step(ck, b0, b1, wait_p=True)
            step(ck + 1, b1, b0, wait_p=True)
            return carry

        lax.fori_loop(0, (_SNCK - 2) // 2, pair, 0)

        if _SNCK % 2 == 1:
            step(_SNCK - 1, b0, b1, wait_p=True)
            add(b0)
            wait_add(b1)
            wait_add(b0)
        else:
            add(b1)
            wait_add(b0)
            wait_add(b1)

        plsc.subcore_barrier()
        pltpu.sync_copy(
            acc_sh.at[pl.ds(s * _SC_NPC, _SC_NPC)],
            agg_hbm.at[pl.ds(s * _SC_NPC, _SC_NPC), pl.ds(col, _HD)])

    return sc_scatter_add


def _sc_scatter_add(e_new, dst, zero):
    return _make_sc_scatter_add()(e_new, dst, zero)[:N]


def kernel(x, edge_index, edge_attr, params):
    src = edge_index[0].astype(jnp.int32)
    dst = edge_index[1].astype(jnp.int32)
    dstN = dst + N
    zero = jnp.zeros((_SC_NPC, _HD), jnp.float32)

    for c in range(len(params['convs'])):
        ep = params['convs'][c]['edge_mlp']
        npar = params['convs'][c]['node_mlp']
        (w1, b1), (w2, b2) = ep
        (wn1, bn1), (wn2, bn2) = npar
        w_all = jnp.concatenate([w1[:D], w1[D:2 * D], wn1[:D]], axis=1)
        p = _proj(x, w_all)
        gs, gd = _sc_gather2(p.reshape(3 * N, D), src, dstN)
        e_new = _edge2(gs, gd, edge_attr, w1[2 * D:], b1.reshape(1, D), w2,
                       b2.reshape(1, D))
        agg = _sc_scatter_add(e_new, dst, zero)
        x = _node2(p, agg, x, wn1[D:], bn1.reshape(1, D), wn2,
                   bn2.reshape(1, D))
        edge_attr = e_new

    (wo1, bo1), (wo2, bo2), (wo3, bo3) = params['out']
    out_dim = wo3.shape[1]
    w3p = jnp.pad(wo3, ((0, 0), (0, 128 - out_dim)))
    b3p = jnp.pad(bo3, ((0, 128 - out_dim),))
    o = _out_mlp(x, wo1, bo1.reshape(1, D), wo2, bo2.reshape(1, D),
                 w3p, b3p.reshape(1, 128))
    return o[:, :out_dim]


# speedup vs baseline: 1.5833x; 1.0150x over previous
import functools

import jax
import jax.numpy as jnp
from jax import lax
from jax.experimental import pallas as pl
from jax.experimental.pallas import tpu as pltpu
from jax.experimental.pallas import tpu_sc as plsc

N = 10000
E = 160000
D = 256

NC = 2
NS = 16
NW = NC * NS


_BN = 2000
_BE = 2000


def _full(shape):
    return pl.BlockSpec(shape, lambda i: (0,) * len(shape))


def _rows(shape):
    return pl.BlockSpec(shape, lambda i: (i,) + (0,) * (len(shape) - 1))


def _proj_body(x_ref, w_ref, out_ref):
    d = jnp.dot(x_ref[...], w_ref[...], preferred_element_type=jnp.float32)
    out_ref[0] = d[:, :D]
    out_ref[1] = d[:, D:2 * D]
    out_ref[2] = d[:, 2 * D:]


def _proj(x, w):
    return pl.pallas_call(
        _proj_body,
        grid=(N // _BN,),
        in_specs=[_rows((_BN, D)), _full((D, 3 * D))],
        out_specs=pl.BlockSpec((3, _BN, D), lambda i: (0, i, 0)),
        out_shape=jax.ShapeDtypeStruct((3, N, D), jnp.float32),
    )(x, w)


def _edge2_body(gs_ref, gd_ref, e_ref, w1e_ref, b1_ref, w2_ref, b2_ref,
                out_ref):
    h = jnp.maximum((gs_ref[...] + gd_ref[...]
                     + jnp.dot(e_ref[...], w1e_ref[...],
                               preferred_element_type=jnp.float32))
                    + b1_ref[...],
                    0.0)
    out_ref[...] = (jnp.dot(h, w2_ref[...],
                            preferred_element_type=jnp.float32)
                    + b2_ref[...] + e_ref[...])


def _edge2(g_all, e, w1e, b1, w2, b2):
    nblk = E // _BE
    return pl.pallas_call(
        _edge2_body,
        grid=(nblk,),
        in_specs=[pl.BlockSpec((_BE, D), lambda i: (i, 0)),
                  pl.BlockSpec((_BE, D), lambda i: (i + nblk, 0)),
                  _rows((_BE, D)),
                  _full((D, D)), _full((1, D)), _full((D, D)),
                  _full((1, D))],
        out_specs=_rows((_BE, D)),
        out_shape=jax.ShapeDtypeStruct((E, D), jnp.float32),
    )(g_all, g_all, e, w1e, b1, w2, b2)


def _node2_body(p_ref, agg_ref, x_ref, w1a_ref, b1_ref, w2_ref, b2_ref,
                out_ref):
    t = (p_ref[0]
         + jnp.dot(agg_ref[...], w1a_ref[...],
                   preferred_element_type=jnp.float32)) + b1_ref[...]
    h = jnp.maximum(t, 0.0)
    out_ref[...] = (jnp.dot(h, w2_ref[...],
                            preferred_element_type=jnp.float32)
                    + b2_ref[...] + x_ref[...])


def _node2(p, agg, x, w1a, b1, w2, b2):
    return pl.pallas_call(
        _node2_body,
        grid=(N // _BN,),
        in_specs=[pl.BlockSpec((1, _BN, D), lambda i: (2, i, 0)),
                  _rows((_BN, D)), _rows((_BN, D)),
                  _full((D, D)), _full((1, D)), _full((D, D)), _full((1, D))],
        out_specs=_rows((_BN, D)),
        out_shape=jax.ShapeDtypeStruct((N, D), jnp.float32),
    )(p, agg, x, w1a, b1, w2, b2)


def _out_mlp_body(x_ref, w1_ref, b1_ref, w2_ref, b2_ref, w3_ref, b3_ref,
                  out_ref):
    h = jnp.maximum(jnp.dot(x_ref[...], w1_ref[...],
                            preferred_element_type=jnp.float32) + b1_ref[...],
                    0.0)
    h = jnp.maximum(jnp.dot(h, w2_ref[...],
                            preferred_element_type=jnp.float32) + b2_ref[...],
                    0.0)
    out_ref[...] = jnp.dot(h, w3_ref[...],
                           preferred_element_type=jnp.float32) + b3_ref[...]


def _out_mlp(x, w1, b1, w2, b2, w3p, b3p):
    return pl.pallas_call(
        _out_mlp_body,
        grid=(N // _BN,),
        in_specs=[_rows((_BN, D)),
                  _full((D, D)), _full((1, D)),
                  _full((D, D)), _full((1, D)),
                  _full((D, 128)), _full((1, 128))],
        out_specs=_rows((_BN, 128)),
        out_shape=jax.ShapeDtypeStruct((N, 128), jnp.float32),
    )(x, w1, b1, w2, b2, w3p, b3p)


_GC = 200
_GPW = 2 * E // NW
_NCK = _GPW // _GC


@functools.lru_cache(maxsize=None)
def _make_sc_gather():
    mesh = plsc.VectorSubcoreMesh(core_axis_name="c", subcore_axis_name="s")

    @functools.partial(
        pl.kernel,
        out_type=jax.ShapeDtypeStruct((2 * E, D), jnp.float32),
        mesh=mesh,
        scratch_types=[
            pltpu.VMEM((_GC,), jnp.int32),
            pltpu.VMEM((_GC,), jnp.int32),
            pltpu.VMEM((_GC, D), jnp.float32),
            pltpu.VMEM((_GC, D), jnp.float32),
            pltpu.SemaphoreType.DMA,
            pltpu.SemaphoreType.DMA,
            pltpu.SemaphoreType.DMA,
            pltpu.SemaphoreType.DMA,
        ],
    )
    def sc_gather(p_hbm, idx_hbm, g_hbm,
                  i0, i1, r0, r1, sg0, sg1, so0, so1):
        wid = lax.axis_index("s") * NC + lax.axis_index("c")
        base = wid * _GPW

        b0 = (i0, r0, sg0, so0)
        b1 = (i1, r1, sg1, so1)

        def idx_load(c, buf):
            pltpu.sync_copy(idx_hbm.at[pl.ds(base + c * _GC, _GC)], buf[0])

        def gather(buf):
            pltpu.async_copy(p_hbm.at[buf[0]], buf[1], buf[2])

        def step(c, bP, bQ, wait_out_q):
            idx_load(c + 1, bQ)
            if wait_out_q:
                pltpu.make_async_copy(bQ[1], g_hbm.at[pl.ds(0, _GC)],
                                      bQ[3]).wait()
            gather(bQ)
            pltpu.make_async_copy(p_hbm.at[bP[0]], bP[1], bP[2]).wait()
            pltpu.async_copy(bP[1], g_hbm.at[pl.ds(base + c * _GC, _GC)],
                             bP[3])

        idx_load(0, b0)
        gather(b0)
        step(0, b0, b1, wait_out_q=False)

        def pair(i, carry):
            c = 1 + 2 * i
            step(c, b1, b0, wait_out_q=True)
            step(c + 1, b0, b1, wait_out_q=True)
            return carry

        lax.fori_loop(0, (_NCK - 2) // 2, pair, 0)

        c_last = _NCK - 1
        pltpu.make_async_copy(p_hbm.at[i1], r1, sg1).wait()
        pltpu.async_copy(r1, g_hbm.at[pl.ds(base + c_last * _GC, _GC)], so1)
        pltpu.make_async_copy(r0, g_hbm.at[pl.ds(0, _GC)], so0).wait()
        pltpu.make_async_copy(r1, g_hbm.at[pl.ds(0, _GC)], so1).wait()

    return sc_gather


def _sc_gather(p, idx_all):
    return _make_sc_gather()(p, idx_all)


_NP = 10240
_SC_NPC = _NP // NS
_SCC = 80
_EPS = E // NS
_SNCK = _EPS // _SCC
_HD = D // NC


@functools.lru_cache(maxsize=None)
def _make_sc_scatter_add():
    mesh = plsc.VectorSubcoreMesh(core_axis_name="c", subcore_axis_name="s")

    @functools.partial(
        pl.kernel,
        out_type=jax.ShapeDtypeStruct((_NP, D), jnp.float32),
        mesh=mesh,
        scratch_types=[
            pltpu.VMEM((_SCC,), jnp.int32),
            pltpu.VMEM((_SCC,), jnp.int32),
            pltpu.VMEM((_SCC, _HD), jnp.float32),
            pltpu.VMEM((_SCC, _HD), jnp.float32),
            pltpu.VMEM_SHARED((_NP, _HD), jnp.float32),
            pltpu.SemaphoreType.DMA,
            pltpu.SemaphoreType.DMA,
            pltpu.SemaphoreType.DMA,
            pltpu.SemaphoreType.DMA,
        ],
    )
    def sc_scatter_add(enew_hbm, dst_hbm, zero_hbm, agg_hbm,
                       di0, di1, rw0, rw1, acc_sh, sr0, sr1, sa0, sa1):
        c = lax.axis_index("c")
        s = lax.axis_index("s")
        pltpu.sync_copy(zero_hbm, acc_sh.at[pl.ds(s * _SC_NPC, _SC_NPC)])
        plsc.subcore_barrier()

        col = c * _HD
        base = s * _EPS
        b0 = (di0, rw0, sr0, sa0)
        b1 = (di1, rw1, sr1, sa1)

        def load(ck, buf):
            off = base + ck * _SCC
            pltpu.sync_copy(dst_hbm.at[pl.ds(off, _SCC)], buf[0])
            pltpu.async_copy(enew_hbm.at[pl.ds(off, _SCC), pl.ds(col, _HD)],
                             buf[1], buf[2])

        def add(buf):
            di, rw, sr, sa = buf
            pltpu.make_async_copy(
                enew_hbm.at[pl.ds(0, _SCC), pl.ds(col, _HD)], rw, sr).wait()
            pltpu.async_copy(rw, acc_sh.at[di], sa, add=True)

        def wait_add(buf):
            pltpu.make_async_copy(buf[1], acc_sh.at[buf[0]], buf[3]).wait()

        def step(ck, bP, bQ, wait_p):
            if wait_p:
                wait_add(bP)
            load(ck, bP)
            add(bQ)

        load(0, b0)
        step(1, b1, b0, wait_p=False)

        def pair(i, carry):
            ck = 2 + 2 * i
            step(ck, b0, b1, wait_p=True)
            step(ck + 1, b1, b0, wait_p=True)
            return carry

        lax.fori_loop(0, (_SNCK - 2) // 2, pair, 0)

        if _SNCK % 2 == 1:
            step(_SNCK - 1, b0, b1, wait_p=True)
            add(b0)
            wait_add(b1)
            wait_add(b0)
        else:
            add(b1)
            wait_add(b0)
            wait_add(b1)

        plsc.subcore_barrier()
        pltpu.sync_copy(
            acc_sh.at[pl.ds(s * _SC_NPC, _SC_NPC)],
            agg_hbm.at[pl.ds(s * _SC_NPC, _SC_NPC), pl.ds(col, _HD)])

    return sc_scatter_add


def _sc_scatter_add(e_new, dst, zero):
    return _make_sc_scatter_add()(e_new, dst, zero)[:N]


def kernel(x, edge_index, edge_attr, params):
    src = edge_index[0].astype(jnp.int32)
    dst = edge_index[1].astype(jnp.int32)
    idx_all = jnp.concatenate([src, dst + N])
    zero = jnp.zeros((_SC_NPC, _HD), jnp.float32)

    for c in range(len(params['convs'])):
        ep = params['convs'][c]['edge_mlp']
        npar = params['convs'][c]['node_mlp']
        (w1, b1), (w2, b2) = ep
        (wn1, bn1), (wn2, bn2) = npar
        w_all = jnp.concatenate([w1[:D], w1[D:2 * D], wn1[:D]], axis=1)
        p = _proj(x, w_all)
        g = _sc_gather(p.reshape(3 * N, D), idx_all)
        e_new = _edge2(g, edge_attr, w1[2 * D:], b1.reshape(1, D), w2,
                       b2.reshape(1, D))
        agg = _sc_scatter_add(e_new, dst, zero)
        x = _node2(p, agg, x, wn1[D:], bn1.reshape(1, D), wn2,
                   bn2.reshape(1, D))
        edge_attr = e_new

    (wo1, bo1), (wo2, bo2), (wo3, bo3) = params['out']
    out_dim = wo3.shape[1]
    w3p = jnp.pad(wo3, ((0, 0), (0, 128 - out_dim)))
    b3p = jnp.pad(bo3, ((0, 128 - out_dim),))
    o = _out_mlp(x, wo1, bo1.reshape(1, D), wo2, bo2.reshape(1, D),
                 w3p, b3p.reshape(1, 128))
    return o[:, :out_dim]
